# baseline (speedup 1.0000x reference)
import functools

import jax
import jax.numpy as jnp
from jax import lax
from jax.experimental import pallas as pl
from jax.experimental.pallas import tpu as pltpu
from jax.experimental.pallas import tpu_sc as plsc

B, P, F = 1024, 128, 6
K = 3

NC, NS = 2, 16
NW = NC * NS
NE = B * K * P
RPW = NE // NW
CH = 2048
GD = 8

CA = 32
CB = 64


def _knn_body(x_ref, cr_ref, idx_ref):
    x = x_ref[...]
    cc = x[:, :, 1:3]
    cr = cr_ref[...]
    iota_s = lax.broadcasted_iota(jnp.int32, (CA, P, P), 1)
    iota_l = lax.broadcasted_iota(jnp.int32, (CA, P, P), 2)
    dx = cc[:, :, 0:1] - cr[:, 0:1, :]
    dy = cc[:, :, 1:2] - cr[:, 1:2, :]
    d2 = dx * dx + dy * dy
    d2 = d2 + jnp.where(iota_s == iota_l, jnp.float32(1e9), jnp.float32(0.0))
    base = (pl.program_id(0) * CA
            + lax.broadcasted_iota(jnp.int32, (CA, 1, P), 0)) * P
    cols = []
    for k in range(K):
        m = jnp.min(d2, axis=1, keepdims=True)
        idx = jnp.min(jnp.where(d2 == m, iota_s, P), axis=1, keepdims=True)
        cols.append(idx + base)
        if k < K - 1:
            d2 = jnp.where(iota_s == idx, jnp.float32(2e9), d2)
    idx_ref[...] = jnp.concatenate(cols, axis=1).reshape(CA * K, P)


def _sc_gather_body(idx_hbm, x16_hbm, out_hbm, idx_v, rows_v, sem):
    wid = lax.axis_index("s") * NC + lax.axis_index("c")
    for t in range(RPW // CH):
        base = wid * RPW + t * CH
        pltpu.sync_copy(idx_hbm.at[pl.ds(base, CH)], idx_v)
        pltpu.async_copy(x16_hbm.at[idx_v], rows_v, sem).wait()
        pltpu.sync_copy(rows_v, out_hbm.at[pl.ds(base, CH)])


def _mlp_body(x_ref, xj_ref, Wcat_ref, b1_ref, W2_ref, b2_ref,
              W3_ref, b3_ref, out_ref):
    x = x_ref[...]
    xj = xj_ref[...]
    xb = jnp.broadcast_to(x[:, None], (CB, K, P, F))
    cat = jnp.concatenate([xb, xj], axis=-1)
    cat2 = cat.reshape(CB * K * P, F + GD)
    h1 = jnp.maximum(
        jnp.dot(cat2, Wcat_ref[...],
                preferred_element_type=jnp.float32) + b1_ref[...], 0.0)
    h2 = jnp.maximum(
        jnp.dot(h1, W2_ref[...],
                preferred_element_type=jnp.float32) + b2_ref[...], 0.0)
    h3 = h2.reshape(CB, K * P, 32)
    pooled = jnp.sum(h3, axis=1, keepdims=True) * jnp.float32(1.0 / (K * P))
    logits = jnp.einsum("cps,so->cpo", pooled, W3_ref[...],
                        preferred_element_type=jnp.float32) + b3_ref[...]
    z = logits - jnp.max(logits, axis=2, keepdims=True)
    e = jnp.exp(z)
    out_ref[...] = e / jnp.sum(e, axis=2, keepdims=True)


@jax.jit
def kernel(x, W1, b1, W2, b2, W3, b3):
    x8t = jnp.pad(x, ((0, 0), (0, 0), (0, GD - F))).reshape(
        B * P * GD // 128, 128)
    Wcat = jnp.concatenate(
        [W1[:F] - W1[F:], W1[F:], jnp.zeros((GD - F, 32), W1.dtype)], axis=0)
    b1r = b1.reshape(1, 32)
    b2r = b2.reshape(1, 32)
    b3r = b3.reshape(1, 1, 2)

    cRT = x[:, :, 1:3].transpose(0, 2, 1)
    idxg = pl.pallas_call(
        _knn_body,
        grid=(B // CA,),
        in_specs=[
            pl.BlockSpec((CA, P, F), lambda i: (i, 0, 0)),
            pl.BlockSpec((CA, 2, P), lambda i: (i, 0, 0)),
        ],
        out_specs=pl.BlockSpec((CA * K, P), lambda i: (i, 0)),
        out_shape=jax.ShapeDtypeStruct((B * K, P), jnp.int32),
    )(x, cRT)

    mesh = plsc.VectorSubcoreMesh(core_axis_name="c", subcore_axis_name="s",
                                  num_cores=NC, num_subcores=NS)
    xj_flat = pl.kernel(
        _sc_gather_body,
        out_type=jax.ShapeDtypeStruct((NE, GD), jnp.float32),
        mesh=mesh,
        scratch_types=[
            pltpu.VMEM((CH,), jnp.int32),
            pltpu.VMEM((CH, GD), jnp.float32),
            pltpu.SemaphoreType.DMA,
        ],
        compiler_params=pltpu.CompilerParams(use_tc_tiling_on_sc=False),
    )(idxg.reshape(NE), x8t.reshape(B * P, GD))
    xj = xj_flat.reshape(B, K, P, GD)

    out = pl.pallas_call(
        _mlp_body,
        grid=(B // CB,),
        in_specs=[
            pl.BlockSpec((CB, P, F), lambda i: (i, 0, 0)),
            pl.BlockSpec((CB, K, P, GD), lambda i: (i, 0, 0, 0)),
            pl.BlockSpec((F + GD, 32), lambda i: (0, 0)),
            pl.BlockSpec((1, 32), lambda i: (0, 0)),
            pl.BlockSpec((32, 32), lambda i: (0, 0)),
            pl.BlockSpec((1, 32), lambda i: (0, 0)),
            pl.BlockSpec((32, 2), lambda i: (0, 0)),
            pl.BlockSpec((1, 1, 2), lambda i: (0, 0, 0)),
        ],
        out_specs=pl.BlockSpec((CB, 1, 2), lambda i: (i, 0, 0)),
        out_shape=jax.ShapeDtypeStruct((B, 1, 2), jnp.float32),
    )(x, xj, Wcat, b1r, W2, b2r, W3, b3r)
    return out.reshape(B, 2)


# speedup vs baseline: 1.0967x; 1.0967x over previous
import functools

import jax
import jax.numpy as jnp
from jax import lax
from jax.experimental import pallas as pl
from jax.experimental.pallas import tpu as pltpu
from jax.experimental.pallas import tpu_sc as plsc

B, P, F = 1024, 128, 6
K = 3

NC, NS = 2, 16
NW = NC * NS
NE = B * K * P
RPW = NE // NW
CH = 2048
GD = 8

CA = 32
CB = 64


def _knn_body(x_ref, cr_ref, idx_ref, x8_ref):
    x = x_ref[...]
    zeros = jnp.zeros((CA, P, GD - F), jnp.float32)
    x8_ref[...] = jnp.concatenate([x, zeros], axis=-1).reshape(CA * P, GD)
    cc = x[:, :, 1:3]
    cr = cr_ref[...]
    iota_s = lax.broadcasted_iota(jnp.int32, (CA, P, P), 1)
    iota_l = lax.broadcasted_iota(jnp.int32, (CA, P, P), 2)
    dx = cc[:, :, 0:1] - cr[:, 0:1, :]
    dy = cc[:, :, 1:2] - cr[:, 1:2, :]
    d2 = dx * dx + dy * dy
    d2 = d2 + jnp.where(iota_s == iota_l, jnp.float32(1e9), jnp.float32(0.0))
    base = (pl.program_id(0) * CA
            + lax.broadcasted_iota(jnp.int32, (CA, 1, P), 0)) * P
    cols = []
    for k in range(K):
        m = jnp.min(d2, axis=1, keepdims=True)
        idx = jnp.min(jnp.where(d2 == m, iota_s, P), axis=1, keepdims=True)
        cols.append(idx + base)
        if k < K - 1:
            d2 = jnp.where(iota_s == idx, jnp.float32(2e9), d2)
    idx_ref[...] = jnp.concatenate(cols, axis=1).reshape(CA * K, P)


def _sc_gather_body(idx_hbm, x16_hbm, out_hbm, idx_v, rows_v, sem):
    wid = lax.axis_index("s") * NC + lax.axis_index("c")
    for t in range(RPW // CH):
        base = wid * RPW + t * CH
        pltpu.sync_copy(idx_hbm.at[pl.ds(base, CH)], idx_v)
        pltpu.async_copy(x16_hbm.at[idx_v], rows_v, sem).wait()
        pltpu.sync_copy(rows_v, out_hbm.at[pl.ds(base, CH)])


def _mlp_body(x_ref, xj_ref, Wcat_ref, b1_ref, W2_ref, b2_ref,
              W3_ref, b3_ref, out_ref):
    x = x_ref[...]
    xj = xj_ref[...]
    xb = jnp.broadcast_to(x[:, None], (CB, K, P, F))
    cat = jnp.concatenate([xb, xj], axis=-1)
    cat2 = cat.reshape(CB * K * P, F + GD)
    h1 = jnp.maximum(
        jnp.dot(cat2, Wcat_ref[...],
                preferred_element_type=jnp.float32) + b1_ref[...], 0.0)
    h2 = jnp.maximum(
        jnp.dot(h1, W2_ref[...],
                preferred_element_type=jnp.float32) + b2_ref[...], 0.0)
    h3 = h2.reshape(CB, K * P, 32)
    pooled = jnp.sum(h3, axis=1, keepdims=True) * jnp.float32(1.0 / (K * P))
    logits = jnp.einsum("cps,so->cpo", pooled, W3_ref[...],
                        preferred_element_type=jnp.float32) + b3_ref[...]
    z = logits - jnp.max(logits, axis=2, keepdims=True)
    e = jnp.exp(z)
    out_ref[...] = e / jnp.sum(e, axis=2, keepdims=True)


@jax.jit
def kernel(x, W1, b1, W2, b2, W3, b3):
    Wcat = jnp.concatenate(
        [W1[:F] - W1[F:], W1[F:], jnp.zeros((GD - F, 32), W1.dtype)], axis=0)
    b1r = b1.reshape(1, 32)
    b2r = b2.reshape(1, 32)
    b3r = b3.reshape(1, 1, 2)

    cRT = x[:, :, 1:3].transpose(0, 2, 1)
    idxg, x8t = pl.pallas_call(
        _knn_body,
        grid=(B // CA,),
        in_specs=[
            pl.BlockSpec((CA, P, F), lambda i: (i, 0, 0)),
            pl.BlockSpec((CA, 2, P), lambda i: (i, 0, 0)),
        ],
        out_specs=[
            pl.BlockSpec((CA * K, P), lambda i: (i, 0)),
            pl.BlockSpec((CA * P, GD), lambda i: (i, 0)),
        ],
        out_shape=[
            jax.ShapeDtypeStruct((B * K, P), jnp.int32),
            jax.ShapeDtypeStruct((B * P, GD), jnp.float32),
        ],
    )(x, cRT)

    mesh = plsc.VectorSubcoreMesh(core_axis_name="c", subcore_axis_name="s",
                                  num_cores=NC, num_subcores=NS)
    xj_flat = pl.kernel(
        _sc_gather_body,
        out_type=jax.ShapeDtypeStruct((NE, GD), jnp.float32),
        mesh=mesh,
        scratch_types=[
            pltpu.VMEM((CH,), jnp.int32),
            pltpu.VMEM((CH, GD), jnp.float32),
            pltpu.SemaphoreType.DMA,
        ],
        compiler_params=pltpu.CompilerParams(use_tc_tiling_on_sc=False),
    )(idxg.reshape(NE), x8t)
    xj = xj_flat.reshape(B, K, P, GD)

    out = pl.pallas_call(
        _mlp_body,
        grid=(B // CB,),
        in_specs=[
            pl.BlockSpec((CB, P, F), lambda i: (i, 0, 0)),
            pl.BlockSpec((CB, K, P, GD), lambda i: (i, 0, 0, 0)),
            pl.BlockSpec((F + GD, 32), lambda i: (0, 0)),
            pl.BlockSpec((1, 32), lambda i: (0, 0)),
            pl.BlockSpec((32, 32), lambda i: (0, 0)),
            pl.BlockSpec((1, 32), lambda i: (0, 0)),
            pl.BlockSpec((32, 2), lambda i: (0, 0)),
            pl.BlockSpec((1, 1, 2), lambda i: (0, 0, 0)),
        ],
        out_specs=pl.BlockSpec((CB, 1, 2), lambda i: (i, 0, 0)),
        out_shape=jax.ShapeDtypeStruct((B, 1, 2), jnp.float32),
    )(x, xj, Wcat, b1r, W2, b2r, W3, b3r)
    return out.reshape(B, 2)


# speedup vs baseline: 1.1262x; 1.0269x over previous
import functools

import jax
import jax.numpy as jnp
from jax import lax
from jax.experimental import pallas as pl
from jax.experimental.pallas import tpu as pltpu
from jax.experimental.pallas import tpu_sc as plsc

B, P, F = 1024, 128, 6
K = 3

NC, NS = 2, 16
NW = NC * NS
NE = B * K * P
RPW = NE // NW
CH = 2048
GD = 8

CA = 32
CB = 64


def _knn_body(x_ref, cr_ref, idx_ref, x8_ref):
    x = x_ref[...]
    zeros = jnp.zeros((CA, P, GD - F), jnp.float32)
    x8_ref[...] = jnp.concatenate([x, zeros], axis=-1).reshape(CA * P, GD)
    cc = x[:, :, 1:3]
    cr = cr_ref[...]
    iota_s = lax.broadcasted_iota(jnp.int32, (CA, P, P), 1)
    iota_l = lax.broadcasted_iota(jnp.int32, (CA, P, P), 2)
    dx = cc[:, :, 0:1] - cr[:, 0:1, :]
    dy = cc[:, :, 1:2] - cr[:, 1:2, :]
    d2 = dx * dx + dy * dy
    d2 = d2 + jnp.where(iota_s == iota_l, jnp.float32(1e9), jnp.float32(0.0))
    base = (pl.program_id(0) * CA
            + lax.broadcasted_iota(jnp.int32, (CA, 1, P), 0)) * P
    cols = []
    for k in range(K):
        m = jnp.min(d2, axis=1, keepdims=True)
        idx = jnp.min(jnp.where(d2 == m, iota_s, P), axis=1, keepdims=True)
        cols.append(idx + base)
        if k < K - 1:
            d2 = jnp.where(iota_s == idx, jnp.float32(2e9), d2)
    idx_ref[...] = jnp.concatenate(cols, axis=1).reshape(CA * K, P)


def _sc_gather_body(ne, idx_hbm, x16_hbm, out_hbm, idx_v, rows_v, sem):
    wid = lax.axis_index("s") * NC + lax.axis_index("c")
    rpw = ne // NW
    for t in range(rpw // CH):
        base = wid * rpw + t * CH
        pltpu.sync_copy(idx_hbm.at[pl.ds(base, CH)], idx_v)
        pltpu.async_copy(x16_hbm.at[idx_v], rows_v, sem).wait()
        pltpu.sync_copy(rows_v, out_hbm.at[pl.ds(base, CH)])


def _mlp_body(x_ref, xj_ref, Wcat_ref, b1_ref, W2_ref, b2_ref,
              W3_ref, b3_ref, out_ref):
    x = x_ref[...]
    xj = xj_ref[...]
    xb = jnp.broadcast_to(x[:, None], (CB, K, P, F))
    cat = jnp.concatenate([xb, xj], axis=-1)
    cat2 = cat.reshape(CB * K * P, F + GD)
    h1 = jnp.maximum(
        jnp.dot(cat2, Wcat_ref[...],
                preferred_element_type=jnp.float32) + b1_ref[...], 0.0)
    h2 = jnp.maximum(
        jnp.dot(h1, W2_ref[...],
                preferred_element_type=jnp.float32) + b2_ref[...], 0.0)
    h3 = h2.reshape(CB, K * P, 32)
    pooled = jnp.sum(h3, axis=1, keepdims=True) * jnp.float32(1.0 / (K * P))
    logits = jnp.einsum("cps,so->cpo", pooled, W3_ref[...],
                        preferred_element_type=jnp.float32) + b3_ref[...]
    z = logits - jnp.max(logits, axis=2, keepdims=True)
    e = jnp.exp(z)
    out_ref[...] = e / jnp.sum(e, axis=2, keepdims=True)


@jax.jit
def kernel(x, W1, b1, W2, b2, W3, b3):
    Wcat = jnp.concatenate(
        [W1[:F] - W1[F:], W1[F:], jnp.zeros((GD - F, 32), W1.dtype)], axis=0)
    b1r = b1.reshape(1, 32)
    b2r = b2.reshape(1, 32)
    b3r = b3.reshape(1, 1, 2)
    cRT = x[:, :, 1:3].transpose(0, 2, 1)

    mesh = plsc.VectorSubcoreMesh(core_axis_name="c", subcore_axis_name="s",
                                  num_cores=NC, num_subcores=NS)

    def stage_a(xh, crh, hb):
        return pl.pallas_call(
            _knn_body,
            grid=(hb // CA,),
            in_specs=[
                pl.BlockSpec((CA, P, F), lambda i: (i, 0, 0)),
                pl.BlockSpec((CA, 2, P), lambda i: (i, 0, 0)),
            ],
            out_specs=[
                pl.BlockSpec((CA * K, P), lambda i: (i, 0)),
                pl.BlockSpec((CA * P, GD), lambda i: (i, 0)),
            ],
            out_shape=[
                jax.ShapeDtypeStruct((hb * K, P), jnp.int32),
                jax.ShapeDtypeStruct((hb * P, GD), jnp.float32),
            ],
        )(xh, crh)

    def stage_sc(idxg, x8t, hb):
        ne = hb * K * P
        return pl.kernel(
            functools.partial(_sc_gather_body, ne),
            out_type=jax.ShapeDtypeStruct((ne, GD), jnp.float32),
            mesh=mesh,
            scratch_types=[
                pltpu.VMEM((CH,), jnp.int32),
                pltpu.VMEM((CH, GD), jnp.float32),
                pltpu.SemaphoreType.DMA,
            ],
            compiler_params=pltpu.CompilerParams(use_tc_tiling_on_sc=False),
        )(idxg.reshape(ne), x8t)

    def stage_b(xh, xj_flat, hb):
        xj = xj_flat.reshape(hb, K, P, GD)
        return pl.pallas_call(
            _mlp_body,
            grid=(hb // CB,),
            in_specs=[
                pl.BlockSpec((CB, P, F), lambda i: (i, 0, 0)),
                pl.BlockSpec((CB, K, P, GD), lambda i: (i, 0, 0, 0)),
                pl.BlockSpec((F + GD, 32), lambda i: (0, 0)),
                pl.BlockSpec((1, 32), lambda i: (0, 0)),
                pl.BlockSpec((32, 32), lambda i: (0, 0)),
                pl.BlockSpec((1, 32), lambda i: (0, 0)),
                pl.BlockSpec((32, 2), lambda i: (0, 0)),
                pl.BlockSpec((1, 1, 2), lambda i: (0, 0, 0)),
            ],
            out_specs=pl.BlockSpec((CB, 1, 2), lambda i: (i, 0, 0)),
            out_shape=jax.ShapeDtypeStruct((hb, 1, 2), jnp.float32),
        )(xh, xj, Wcat, b1r, W2, b2r, W3, b3r)

    HB = B // 2
    outs = []
    halves = []
    for h in range(2):
        sl = slice(h * HB, (h + 1) * HB)
        halves.append(stage_a(x[sl], cRT[sl], HB))
    gathered = [stage_sc(idxg, x8t, HB) for idxg, x8t in halves]
    for h in range(2):
        sl = slice(h * HB, (h + 1) * HB)
        outs.append(stage_b(x[sl], gathered[h], HB))
    out = jnp.concatenate(outs, axis=0)
    return out.reshape(B, 2)


# speedup vs baseline: 1.1285x; 1.0021x over previous
import functools

import jax
import jax.numpy as jnp
from jax import lax
from jax.experimental import pallas as pl
from jax.experimental.pallas import tpu as pltpu
from jax.experimental.pallas import tpu_sc as plsc

B, P, F = 1024, 128, 6
K = 3

NC, NS = 2, 16
NW = NC * NS
NE = B * K * P
RPW = NE // NW
CH = 2048
GD = 8

CA = 32
CB = 64


def _knn_body(x_ref, cr_ref, idx_ref, x8_ref):
    x = x_ref[...]
    zeros = jnp.zeros((CA, P, GD - F), jnp.float32)
    x8_ref[...] = jnp.concatenate([x, zeros], axis=-1).reshape(CA * P, GD)
    cc = x[:, :, 1:3]
    cr = cr_ref[...]
    iota_s = lax.broadcasted_iota(jnp.int32, (CA, P, P), 1)
    iota_l = lax.broadcasted_iota(jnp.int32, (CA, P, P), 2)
    dx = cc[:, :, 0:1] - cr[:, 0:1, :]
    dy = cc[:, :, 1:2] - cr[:, 1:2, :]
    d2 = dx * dx + dy * dy
    d2 = d2 + jnp.where(iota_s == iota_l, jnp.float32(1e9), jnp.float32(0.0))
    base = (pl.program_id(0) * CA
            + lax.broadcasted_iota(jnp.int32, (CA, 1, P), 0)) * P
    cols = []
    for k in range(K):
        m = jnp.min(d2, axis=1, keepdims=True)
        idx = jnp.min(jnp.where(d2 == m, iota_s, P), axis=1, keepdims=True)
        cols.append(idx + base)
        if k < K - 1:
            d2 = jnp.where(iota_s == idx, jnp.float32(2e9), d2)
    idx_ref[...] = jnp.concatenate(cols, axis=1).reshape(CA * K, P)


def _sc_gather_body(ne, idx_hbm, x16_hbm, out_hbm, idx_v, rows_v, sem):
    wid = lax.axis_index("s") * NC + lax.axis_index("c")
    rpw = ne // NW
    for t in range(rpw // CH):
        base = wid * rpw + t * CH
        pltpu.sync_copy(idx_hbm.at[pl.ds(base, CH)], idx_v)
        pltpu.async_copy(x16_hbm.at[idx_v], rows_v, sem).wait()
        pltpu.sync_copy(rows_v, out_hbm.at[pl.ds(base, CH)])


def _mlp_body(x8p_ref, xj_ref, W1a_ref, W1b_ref, b1_ref, W2_ref, b2_ref,
              W3_ref, b3_ref, out_ref):
    selfp = x8p_ref[...]
    AP = jnp.dot(selfp, W1a_ref[...],
                 preferred_element_type=jnp.float32) + b1_ref[...]
    xjp = xj_ref[...]
    G = jnp.dot(xjp, W1b_ref[...],
                preferred_element_type=jnp.float32)
    h1 = jnp.maximum(G.reshape(CB, K, 8, 512)
                     + AP.reshape(CB, 1, 8, 512), 0.0)
    h2 = jnp.maximum(
        jnp.dot(h1.reshape(CB * K * 8, 512), W2_ref[...],
                preferred_element_type=jnp.float32) + b2_ref[...], 0.0)
    ps = jnp.sum(h2.reshape(CB, K * 8, 512), axis=1)
    pooled = ps[:, 0:32]
    for sslot in range(1, 16):
        pooled = pooled + ps[:, 32 * sslot:32 * (sslot + 1)]
    pooled = pooled * jnp.float32(1.0 / (K * P))
    logits = jnp.dot(pooled, W3_ref[...],
                     preferred_element_type=jnp.float32) + b3_ref[...]
    z = logits - jnp.max(logits, axis=1, keepdims=True)
    e = jnp.exp(z)
    out_ref[...] = e / jnp.sum(e, axis=1, keepdims=True)


@jax.jit
def kernel(x, W1, b1, W2, b2, W3, b3):
    eye16 = jnp.eye(16, dtype=jnp.float32)
    W1ab8 = jnp.pad(W1[:F] - W1[F:], ((0, GD - F), (0, 0)))
    W1b8 = jnp.pad(W1[F:], ((0, GD - F), (0, 0)))
    W1a_blk = jnp.kron(eye16, W1ab8)
    W1b_blk = jnp.kron(eye16, W1b8)
    W2_blk = jnp.kron(eye16, W2)
    b1t = jnp.tile(b1, 16).reshape(1, 512)
    b2t = jnp.tile(b2, 16).reshape(1, 512)
    b3r = b3.reshape(1, 2)
    cRT = x[:, :, 1:3].transpose(0, 2, 1)
    x8p = jnp.pad(x, ((0, 0), (0, 0), (0, GD - F))).reshape(B * 8, 128)

    mesh = plsc.VectorSubcoreMesh(core_axis_name="c", subcore_axis_name="s",
                                  num_cores=NC, num_subcores=NS)

    def stage_a(xh, crh, hb):
        return pl.pallas_call(
            _knn_body,
            grid=(hb // CA,),
            in_specs=[
                pl.BlockSpec((CA, P, F), lambda i: (i, 0, 0)),
                pl.BlockSpec((CA, 2, P), lambda i: (i, 0, 0)),
            ],
            out_specs=[
                pl.BlockSpec((CA * K, P), lambda i: (i, 0)),
                pl.BlockSpec((CA * P, GD), lambda i: (i, 0)),
            ],
            out_shape=[
                jax.ShapeDtypeStruct((hb * K, P), jnp.int32),
                jax.ShapeDtypeStruct((hb * P, GD), jnp.float32),
            ],
        )(xh, crh)

    def stage_sc(idxg, x8t, hb):
        ne = hb * K * P
        return pl.kernel(
            functools.partial(_sc_gather_body, ne),
            out_type=jax.ShapeDtypeStruct((ne, GD), jnp.float32),
            mesh=mesh,
            scratch_types=[
                pltpu.VMEM((CH,), jnp.int32),
                pltpu.VMEM((CH, GD), jnp.float32),
                pltpu.SemaphoreType.DMA,
            ],
            compiler_params=pltpu.CompilerParams(use_tc_tiling_on_sc=False),
        )(idxg.reshape(ne), x8t)

    def stage_b(x8ph, xj_flat, hb):
        xjp = xj_flat.reshape(hb * K * P * GD // 128, 128)
        return pl.pallas_call(
            _mlp_body,
            grid=(hb // CB,),
            in_specs=[
                pl.BlockSpec((CB * 8, 128), lambda i: (i, 0)),
                pl.BlockSpec((CB * K * 8, 128), lambda i: (i, 0)),
                pl.BlockSpec((128, 512), lambda i: (0, 0)),
                pl.BlockSpec((128, 512), lambda i: (0, 0)),
                pl.BlockSpec((1, 512), lambda i: (0, 0)),
                pl.BlockSpec((512, 512), lambda i: (0, 0)),
                pl.BlockSpec((1, 512), lambda i: (0, 0)),
                pl.BlockSpec((32, 2), lambda i: (0, 0)),
                pl.BlockSpec((1, 2), lambda i: (0, 0)),
            ],
            out_specs=pl.BlockSpec((CB, 2), lambda i: (i, 0)),
            out_shape=jax.ShapeDtypeStruct((hb, 2), jnp.float32),
        )(x8ph, xjp, W1a_blk, W1b_blk, b1t, W2_blk, b2t, W3, b3r)

    HB = B // 2
    outs = []
    halves = []
    for h in range(2):
        sl = slice(h * HB, (h + 1) * HB)
        halves.append(stage_a(x[sl], cRT[sl], HB))
    gathered = [stage_sc(idxg, x8t, HB) for idxg, x8t in halves]
    for h in range(2):
        slp = slice(h * HB * 8, (h + 1) * HB * 8)
        outs.append(stage_b(x8p[slp], gathered[h], HB))
    return jnp.concatenate(outs, axis=0)


# speedup vs baseline: 1.1379x; 1.0083x over previous
import functools

import jax
import jax.numpy as jnp
from jax import lax
from jax.experimental import pallas as pl
from jax.experimental.pallas import tpu as pltpu
from jax.experimental.pallas import tpu_sc as plsc

B, P, F = 1024, 128, 6
K = 3

NC, NS = 2, 16
NW = NC * NS
NE = B * K * P
RPW = NE // NW
CH = 2048
GD = 8

CA = 32
CB = 64


def _knn_body(x_ref, cr_ref, idx_ref, x8_ref):
    x = x_ref[...]
    zeros = jnp.zeros((CA, P, GD - F), jnp.float32)
    x8_ref[...] = jnp.concatenate([x, zeros], axis=-1).reshape(CA * P, GD)
    cc = x[:, :, 1:3]
    cr = cr_ref[...]
    iota_s = lax.broadcasted_iota(jnp.int32, (CA, P, P), 1)
    iota_l = lax.broadcasted_iota(jnp.int32, (CA, P, P), 2)
    dx = cc[:, :, 0:1] - cr[:, 0:1, :]
    dy = cc[:, :, 1:2] - cr[:, 1:2, :]
    d2 = dx * dx + dy * dy
    d2 = d2 + jnp.where(iota_s == iota_l, jnp.float32(1e9), jnp.float32(0.0))
    base = (pl.program_id(0) * CA
            + lax.broadcasted_iota(jnp.int32, (CA, 1, P), 0)) * P
    cols = []
    for k in range(K):
        m = jnp.min(d2, axis=1, keepdims=True)
        idx = jnp.min(jnp.where(d2 == m, iota_s, P), axis=1, keepdims=True)
        cols.append(idx + base)
        if k < K - 1:
            d2 = jnp.where(iota_s == idx, jnp.float32(2e9), d2)
    idx_ref[...] = jnp.concatenate(cols, axis=1).reshape(CA * K, P)


def _sc_gather_body(ne, idx_hbm, x16_hbm, out_hbm, idx_v, rows_v, sem):
    wid = lax.axis_index("s") * NC + lax.axis_index("c")
    rpw = ne // NW
    for t in range(rpw // CH):
        base = wid * rpw + t * CH
        pltpu.sync_copy(idx_hbm.at[pl.ds(base, CH)], idx_v)
        pltpu.async_copy(x16_hbm.at[idx_v], rows_v, sem).wait()
        pltpu.sync_copy(rows_v, out_hbm.at[pl.ds(base, CH)])


def _mlp_body(x8p_ref, xj_ref, W1a_ref, W1b_ref, b1_ref, W2_ref, b2_ref,
              W3_ref, b3_ref, out_ref):
    selfp = x8p_ref[...]
    AP = jnp.dot(selfp, W1a_ref[...],
                 preferred_element_type=jnp.float32) + b1_ref[...]
    xjp = xj_ref[...]
    G = jnp.dot(xjp, W1b_ref[...],
                preferred_element_type=jnp.float32)
    h1 = jnp.maximum(G.reshape(CB, K, 8, 512)
                     + AP.reshape(CB, 1, 8, 512), 0.0)
    h2 = jnp.maximum(
        jnp.dot(h1.reshape(CB * K * 8, 512), W2_ref[...],
                preferred_element_type=jnp.float32) + b2_ref[...], 0.0)
    ps = jnp.sum(h2.reshape(CB, K * 8, 512), axis=1)
    pooled = ps[:, 0:32]
    for sslot in range(1, 16):
        pooled = pooled + ps[:, 32 * sslot:32 * (sslot + 1)]
    pooled = pooled * jnp.float32(1.0 / (K * P))
    logits = jnp.dot(pooled, W3_ref[...],
                     preferred_element_type=jnp.float32) + b3_ref[...]
    z = logits - jnp.max(logits, axis=1, keepdims=True)
    e = jnp.exp(z)
    out_ref[...] = e / jnp.sum(e, axis=1, keepdims=True)


@jax.jit
def kernel(x, W1, b1, W2, b2, W3, b3):
    eye16 = jnp.eye(16, dtype=jnp.float32)
    W1ab8 = jnp.pad(W1[:F] - W1[F:], ((0, GD - F), (0, 0)))
    W1b8 = jnp.pad(W1[F:], ((0, GD - F), (0, 0)))
    W1a_blk = jnp.kron(eye16, W1ab8)
    W1b_blk = jnp.kron(eye16, W1b8)
    W2_blk = jnp.kron(eye16, W2)
    b1t = jnp.tile(b1, 16).reshape(1, 512)
    b2t = jnp.tile(b2, 16).reshape(1, 512)
    b3r = b3.reshape(1, 2)
    cRT = x[:, :, 1:3].transpose(0, 2, 1)
    x8p = jnp.pad(x, ((0, 0), (0, 0), (0, GD - F))).reshape(B * 8, 128)

    mesh = plsc.VectorSubcoreMesh(core_axis_name="c", subcore_axis_name="s",
                                  num_cores=NC, num_subcores=NS)

    def stage_a(xh, crh, hb):
        return pl.pallas_call(
            _knn_body,
            grid=(hb // CA,),
            in_specs=[
                pl.BlockSpec((CA, P, F), lambda i: (i, 0, 0)),
                pl.BlockSpec((CA, 2, P), lambda i: (i, 0, 0)),
            ],
            out_specs=[
                pl.BlockSpec((CA * K, P), lambda i: (i, 0)),
                pl.BlockSpec((CA * P, GD), lambda i: (i, 0)),
            ],
            out_shape=[
                jax.ShapeDtypeStruct((hb * K, P), jnp.int32),
                jax.ShapeDtypeStruct((hb * P, GD), jnp.float32),
            ],
        )(xh, crh)

    def stage_sc(idxg, x8t, hb):
        ne = hb * K * P
        return pl.kernel(
            functools.partial(_sc_gather_body, ne),
            out_type=jax.ShapeDtypeStruct((ne, GD), jnp.float32),
            mesh=mesh,
            scratch_types=[
                pltpu.VMEM((CH,), jnp.int32),
                pltpu.VMEM((CH, GD), jnp.float32),
                pltpu.SemaphoreType.DMA,
            ],
            compiler_params=pltpu.CompilerParams(use_tc_tiling_on_sc=False),
        )(idxg.reshape(ne), x8t)

    def stage_b(x8ph, xj_flat, hb):
        xjp = xj_flat.reshape(hb * K * P * GD // 128, 128)
        return pl.pallas_call(
            _mlp_body,
            grid=(hb // CB,),
            in_specs=[
                pl.BlockSpec((CB * 8, 128), lambda i: (i, 0)),
                pl.BlockSpec((CB * K * 8, 128), lambda i: (i, 0)),
                pl.BlockSpec((128, 512), lambda i: (0, 0)),
                pl.BlockSpec((128, 512), lambda i: (0, 0)),
                pl.BlockSpec((1, 512), lambda i: (0, 0)),
                pl.BlockSpec((512, 512), lambda i: (0, 0)),
                pl.BlockSpec((1, 512), lambda i: (0, 0)),
                pl.BlockSpec((32, 2), lambda i: (0, 0)),
                pl.BlockSpec((1, 2), lambda i: (0, 0)),
            ],
            out_specs=pl.BlockSpec((CB, 2), lambda i: (i, 0)),
            out_shape=jax.ShapeDtypeStruct((hb, 2), jnp.float32),
        )(x8ph, xjp, W1a_blk, W1b_blk, b1t, W2_blk, b2t, W3, b3r)

    NSPLIT = 4
    HB = B // NSPLIT
    outs = []
    halves = []
    for h in range(NSPLIT):
        sl = slice(h * HB, (h + 1) * HB)
        halves.append(stage_a(x[sl], cRT[sl], HB))
    gathered = [stage_sc(idxg, x8t, HB) for idxg, x8t in halves]
    for h in range(NSPLIT):
        slp = slice(h * HB * 8, (h + 1) * HB * 8)
        outs.append(stage_b(x8p[slp], gathered[h], HB))
    return jnp.concatenate(outs, axis=0)
